# baseline (speedup 1.0000x reference)
import functools

import jax
import jax.numpy as jnp
from jax import lax
from jax.experimental import pallas as pl
from jax.experimental.pallas import tpu as pltpu
from jax.experimental.pallas import tpu_sc as plsc

B, N, D, K, C = 16, 1024, 16, 20, 64
EPS = 1e-5
NKTOT = B * N * K
NK_CNT = float(B * N * K)
NSC_CNT = float(B * N)
RN = 256
RM = 512
LANES = 128
NROWS = NKTOT // LANES
NW = 32
CHUNKS_PER_W = NROWS // NW
PTS_PER_ROW = LANES // D


def _knn_body(xq_ref, xt_ref, idx_ref):
    b = pl.program_id(0)
    xi_p = xq_ref[0, :, 0:1]
    xi_e = xq_ref[0, :, 1:2]
    xj_p = xt_ref[0:1, :]
    xj_e = xt_ref[1:2, :]
    dx = xj_p - xi_p
    dy = xj_e - xi_e
    s = dx * dx + dy * dy
    jota = lax.broadcasted_iota(jnp.int32, (RN, N), 1)
    key = (lax.bitcast_convert_type(s, jnp.int32) & ~(N - 1)) | jota
    big = jnp.int32(0x7FFFFFFF)
    cols = []
    for k in range(K):
        mk = jnp.min(key, axis=1, keepdims=True)
        cols.append(mk)
        if k + 1 < K:
            key = jnp.where(key == mk, big, key)
    mat = jnp.concatenate(cols, axis=1)
    idx_ref[...] = jnp.transpose((mat & (N - 1)) + b * N)


def _knn_call(x, xt):
    return pl.pallas_call(
        _knn_body,
        grid=(B, N // RN),
        in_specs=[
            pl.BlockSpec((1, RN, D), lambda b, r: (b, r, 0)),
            pl.BlockSpec((8, N), lambda b, r: (0, b)),
        ],
        out_specs=pl.BlockSpec((K, RN), lambda b, r: (0, b * (N // RN) + r)),
        out_shape=jax.ShapeDtypeStruct((K, B * N), jnp.int32),
    )(x, xt)


CH = 512


def _gather_body(xflat_hbm, idx_hbm, out_hbm,
                 x_tile, idx_v, p0, p1, sem0, sem1):
    wid = lax.axis_index("s") * 2 + lax.axis_index("c")
    b = wid // 2
    col0 = wid * CH
    pltpu.sync_copy(xflat_hbm.at[pl.ds(b * N * D, N * D)], x_tile)

    def gather_one(t, pk):
        pltpu.sync_copy(idx_hbm.at[t, pl.ds(col0, CH)], idx_v)
        for c in range(CH // 16):
            iv = idx_v[pl.ds(16 * c, 16)] & (N - 1)
            fbase = iv << 4
            vals = [plsc.load_gather(x_tile, [fbase + d]) for d in range(D)]
            for d in range(D):
                pk[d, pl.ds(16 * c, 16)] = vals[d]

    def pair(u, carry):
        for h, (pk, sem) in enumerate(((p0, sem0), (p1, sem1))):
            t = u * 2 + h

            @pl.when(u >= 1)
            def _():
                pltpu.make_async_copy(
                    pk, out_hbm.at[pl.ds(0, D), pl.ds(col0, CH)], sem).wait()

            gather_one(t, pk)
            pltpu.async_copy(
                pk, out_hbm.at[pl.ds(t * D, D), pl.ds(col0, CH)], sem)
        return carry

    lax.fori_loop(0, K // 2, pair, 0)
    pltpu.make_async_copy(
        p0, out_hbm.at[pl.ds(0, D), pl.ds(col0, CH)], sem0).wait()
    pltpu.make_async_copy(
        p1, out_hbm.at[pl.ds(0, D), pl.ds(col0, CH)], sem1).wait()


def _gather_call(x, idx):
    xflat = x.reshape(B * N * D)
    mesh = plsc.VectorSubcoreMesh(core_axis_name="c", subcore_axis_name="s")
    fn = functools.partial(
        pl.kernel,
        mesh=mesh,
        compiler_params=pltpu.CompilerParams(needs_layout_passes=False),
        out_type=jax.ShapeDtypeStruct((K * D, B * N), jnp.float32),
        scratch_types=[
            pltpu.VMEM((N * D,), jnp.float32),
            pltpu.VMEM((CH,), jnp.int32),
            pltpu.VMEM((D, CH), jnp.float32),
            pltpu.VMEM((D, CH), jnp.float32),
            pltpu.SemaphoreType.DMA,
            pltpu.SemaphoreType.DMA,
        ],
    )(_gather_body)
    return fn(xflat, idx)


def _bn_const(s, q, g_ref, be_ref, cnt):
    mean = s / cnt
    var = q / cnt - mean * mean
    a = g_ref[...] * lax.rsqrt(var + EPS)
    c = be_ref[...] - a * mean
    return a, c


def _dot(a, b):
    return jnp.dot(a.astype(jnp.bfloat16), b.astype(jnp.bfloat16),
                   preferred_element_type=jnp.float32)


def _h0_list(xc_ref, xk_ref, w0b_ref, w0c_ref, b0_ref):
    hc = _dot(w0c_ref[...], xc_ref[...]) + b0_ref[...]
    xk_cat = jnp.concatenate(
        [xk_ref[k * D:(k + 1) * D, :] for k in range(K)], axis=1)
    hc_cat = jnp.concatenate([hc] * K, axis=1)
    return _dot(w0b_ref[...], xk_cat) + hc_cat


def _local_sums(h):
    return (jnp.sum(h, axis=1, keepdims=True),
            jnp.sum(h * h, axis=1, keepdims=True))


def _layer(h, s, q, g, be, w, bias):
    a, c = _bn_const(s, q, g, be, NK_CNT)
    r = jnp.maximum(a * h + c, 0.0)
    return _dot(w[...], r) + bias[...]


def _mlp_body(xc_ref, xk_ref, w0b, w0c, b0, g0, be0, w1, b1, g1, be1,
              w2, b2, g2, be2, wsc, bsc, gsc, besc, out_ref, acc):
    p = pl.program_id(0)

    @pl.when((p == 0) & (pl.program_id(1) == 0))
    def _():
        acc[...] = jnp.zeros_like(acc)

    @pl.when(p == 0)
    def _():
        hs = _h0_list(xc_ref, xk_ref, w0b, w0c, b0)
        s, q = _local_sums(hs)
        scl = _dot(wsc[...], xc_ref[...]) + bsc[...]
        acc[:, 0:1] += s
        acc[:, 1:2] += q
        acc[:, 2:3] += jnp.sum(scl, axis=1, keepdims=True)
        acc[:, 3:4] += jnp.sum(scl * scl, axis=1, keepdims=True)

    @pl.when(p == 1)
    def _():
        hs = _h0_list(xc_ref, xk_ref, w0b, w0c, b0)
        hs = _layer(hs, acc[:, 0:1], acc[:, 1:2], g0, be0, w1, b1)
        s, q = _local_sums(hs)
        acc[:, 4:5] += s
        acc[:, 5:6] += q

    @pl.when(p == 2)
    def _():
        hs = _h0_list(xc_ref, xk_ref, w0b, w0c, b0)
        hs = _layer(hs, acc[:, 0:1], acc[:, 1:2], g0, be0, w1, b1)
        hs = _layer(hs, acc[:, 4:5], acc[:, 5:6], g1, be1, w2, b2)
        s, q = _local_sums(hs)
        acc[:, 6:7] += s
        acc[:, 7:8] += q

    @pl.when(p == 3)
    def _():
        hs = _h0_list(xc_ref, xk_ref, w0b, w0c, b0)
        hs = _layer(hs, acc[:, 0:1], acc[:, 1:2], g0, be0, w1, b1)
        hs = _layer(hs, acc[:, 4:5], acc[:, 5:6], g1, be1, w2, b2)
        a2, c2 = _bn_const(acc[:, 6:7], acc[:, 7:8], g2, be2, NK_CNT)
        r2 = jnp.maximum(a2 * hs + c2, 0.0)
        agg = jnp.zeros((C, RM), jnp.float32)
        for k in range(K):
            agg += r2[:, k * RM:(k + 1) * RM]
        aggr = agg * jnp.float32(1.0 / K)
        scl = _dot(wsc[...], xc_ref[...]) + bsc[...]
        asc, csc = _bn_const(acc[:, 2:3], acc[:, 3:4], gsc, besc, NSC_CNT)
        out_ref[...] = jnp.maximum(aggr + asc * scl + csc, 0.0)


def _cspec(shape):
    return pl.BlockSpec(shape, lambda p, i: tuple(0 for _ in shape))


def kernel(x, W_sc, b_sc, g_sc, be_sc, W0, b0, g0, be0,
           W1, b1, g1, be1, W2, b2, g2, be2):
    xt = jnp.transpose(x.reshape(B * N, D))
    idx = _knn_call(x, xt)
    xk3 = _gather_call(x, idx)

    w0b = W0[:, D:]
    w0c = W0[:, :D] - W0[:, D:]
    col = lambda v: v.reshape(C, 1)
    b0c, bscc = col(b0), col(b_sc)
    b1c, b2c = col(b1), col(b2)
    g0c, be0c = col(g0), col(be0)
    g1c, be1c = col(g1), col(be1)
    g2c, be2c = col(g2), col(be2)
    gscc, bescc = col(g_sc), col(be_sc)

    row_specs = [
        pl.BlockSpec((D, RM), lambda p, i: (0, i)),
        pl.BlockSpec((K * D, RM), lambda p, i: (0, i)),
    ]
    wdc = _cspec((C, D))
    wcc = _cspec((C, C))
    vsp = _cspec((C, 1))

    out_t = pl.pallas_call(
        _mlp_body,
        grid=(4, B * N // RM),
        in_specs=row_specs + [wdc, wdc, vsp, vsp, vsp, wcc, vsp, vsp, vsp,
                              wcc, vsp, vsp, vsp, wdc, vsp, vsp, vsp],
        out_specs=pl.BlockSpec((C, RM), lambda p, i: (0, i)),
        out_shape=jax.ShapeDtypeStruct((C, B * N), jnp.float32),
        scratch_shapes=[pltpu.VMEM((C, 8), jnp.float32)],
    )(xt, xk3, w0b, w0c, b0c, g0c, be0c, W1, b1c, g1c, be1c,
      W2, b2c, g2c, be2c, W_sc, bscc, gscc, bescc)

    return jnp.transpose(out_t).reshape(B, N, C)


# speedup vs baseline: 1.3476x; 1.3476x over previous
import functools

import jax
import jax.numpy as jnp
from jax import lax
from jax.experimental import pallas as pl
from jax.experimental.pallas import tpu as pltpu
from jax.experimental.pallas import tpu_sc as plsc

B, N, D, K, C = 16, 1024, 16, 20, 64
EPS = 1e-5
NKTOT = B * N * K
NK_CNT = float(B * N * K)
NSC_CNT = float(B * N)
RN = 256
RM = 512
LANES = 128
NROWS = NKTOT // LANES
NW = 32
CHUNKS_PER_W = NROWS // NW
PTS_PER_ROW = LANES // D


def _knn_body(xq_ref, xt_ref, idx_ref):
    b = pl.program_id(0)
    xi_p = xq_ref[0, :, 0:1]
    xi_e = xq_ref[0, :, 1:2]
    xj_p = xt_ref[0:1, :]
    xj_e = xt_ref[1:2, :]
    dx = xj_p - xi_p
    dy = xj_e - xi_e
    s = dx * dx + dy * dy
    jota = lax.broadcasted_iota(jnp.int32, (RN, N), 1)
    key = lax.bitcast_convert_type(
        ((lax.bitcast_convert_type(s, jnp.int32) & ~(N - 1)) | jota)
        + jnp.int32(0x00800000),
        jnp.float32)
    big = jnp.float32(3.0e38)
    cols = []
    for k in range(K):
        mk = jnp.min(key, axis=1, keepdims=True)
        cols.append(mk)
        if k + 1 < K:
            key = jnp.where(key == mk, big, key)
    mat = lax.bitcast_convert_type(
        jnp.concatenate(cols, axis=1), jnp.int32)
    idx_ref[...] = jnp.transpose((mat & (N - 1)) + b * N)


def _knn_call(x, xt):
    return pl.pallas_call(
        _knn_body,
        grid=(B, N // RN),
        in_specs=[
            pl.BlockSpec((1, RN, D), lambda b, r: (b, r, 0)),
            pl.BlockSpec((8, N), lambda b, r: (0, b)),
        ],
        out_specs=pl.BlockSpec((K, RN), lambda b, r: (0, b * (N // RN) + r)),
        out_shape=jax.ShapeDtypeStruct((K, B * N), jnp.int32),
    )(x, xt)


CH = 512


def _gather_body(xflat_hbm, idx_hbm, out_hbm,
                 x_tile, idx_v, p0, p1, sem0, sem1):
    wid = lax.axis_index("s") * 2 + lax.axis_index("c")
    b = wid // 2
    col0 = wid * CH
    pltpu.sync_copy(xflat_hbm.at[pl.ds(b * N * D, N * D)], x_tile)

    def gather_one(t, pk):
        pltpu.sync_copy(idx_hbm.at[t, pl.ds(col0, CH)], idx_v)
        for c in range(CH // 16):
            iv = idx_v[pl.ds(16 * c, 16)] & (N - 1)
            fbase = iv << 4
            vals = [plsc.load_gather(x_tile, [fbase + d]) for d in range(D)]
            for d in range(D):
                pk[d, pl.ds(16 * c, 16)] = vals[d]

    def pair(u, carry):
        for h, (pk, sem) in enumerate(((p0, sem0), (p1, sem1))):
            t = u * 2 + h

            @pl.when(u >= 1)
            def _():
                pltpu.make_async_copy(
                    pk, out_hbm.at[pl.ds(0, D), pl.ds(col0, CH)], sem).wait()

            gather_one(t, pk)
            pltpu.async_copy(
                pk, out_hbm.at[pl.ds(t * D, D), pl.ds(col0, CH)], sem)
        return carry

    lax.fori_loop(0, K // 2, pair, 0)
    pltpu.make_async_copy(
        p0, out_hbm.at[pl.ds(0, D), pl.ds(col0, CH)], sem0).wait()
    pltpu.make_async_copy(
        p1, out_hbm.at[pl.ds(0, D), pl.ds(col0, CH)], sem1).wait()


def _gather_call(x, idx):
    xflat = x.reshape(B * N * D)
    mesh = plsc.VectorSubcoreMesh(core_axis_name="c", subcore_axis_name="s")
    fn = functools.partial(
        pl.kernel,
        mesh=mesh,
        compiler_params=pltpu.CompilerParams(needs_layout_passes=False),
        out_type=jax.ShapeDtypeStruct((K * D, B * N), jnp.float32),
        scratch_types=[
            pltpu.VMEM((N * D,), jnp.float32),
            pltpu.VMEM((CH,), jnp.int32),
            pltpu.VMEM((D, CH), jnp.float32),
            pltpu.VMEM((D, CH), jnp.float32),
            pltpu.SemaphoreType.DMA,
            pltpu.SemaphoreType.DMA,
        ],
    )(_gather_body)
    return fn(xflat, idx)


def _bn_const(s, q, g_ref, be_ref, cnt):
    mean = s / cnt
    var = q / cnt - mean * mean
    a = g_ref[...] * lax.rsqrt(var + EPS)
    c = be_ref[...] - a * mean
    return a, c


def _dot(a, b):
    return jnp.dot(a.astype(jnp.bfloat16), b.astype(jnp.bfloat16),
                   preferred_element_type=jnp.float32)


def _h0_list(xc_ref, xk_ref, w0b_ref, w0c_ref, b0_ref):
    hc = _dot(w0c_ref[...], xc_ref[...]) + b0_ref[...]
    out = []
    for k in range(K):
        xkk = xk_ref[k * D:(k + 1) * D, :]
        out.append(_dot(w0b_ref[...], xkk) + hc)
    return out


def _local_sums(hs):
    s = jnp.zeros((C, 1), jnp.float32)
    q = jnp.zeros((C, 1), jnp.float32)
    for h in hs:
        s += jnp.sum(h, axis=1, keepdims=True)
        q += jnp.sum(h * h, axis=1, keepdims=True)
    return s, q


def _layer(hs, s, q, g, be, w, bias):
    a, c = _bn_const(s, q, g, be, NK_CNT)
    out = []
    for h in hs:
        r = jnp.maximum(a * h + c, 0.0)
        out.append(_dot(w[...], r) + bias[...])
    return out


def _mlp_body(xc_ref, xk_ref, w0b, w0c, b0, g0, be0, w1, b1, g1, be1,
              w2, b2, g2, be2, wsc, bsc, gsc, besc, out_ref, acc):
    p = pl.program_id(0)

    @pl.when((p == 0) & (pl.program_id(1) == 0))
    def _():
        acc[...] = jnp.zeros_like(acc)

    @pl.when(p == 0)
    def _():
        hs = _h0_list(xc_ref, xk_ref, w0b, w0c, b0)
        s, q = _local_sums(hs)
        scl = _dot(wsc[...], xc_ref[...]) + bsc[...]
        acc[:, 0:1] += s
        acc[:, 1:2] += q
        acc[:, 2:3] += jnp.sum(scl, axis=1, keepdims=True)
        acc[:, 3:4] += jnp.sum(scl * scl, axis=1, keepdims=True)

    @pl.when(p == 1)
    def _():
        hs = _h0_list(xc_ref, xk_ref, w0b, w0c, b0)
        hs = _layer(hs, acc[:, 0:1], acc[:, 1:2], g0, be0, w1, b1)
        s, q = _local_sums(hs)
        acc[:, 4:5] += s
        acc[:, 5:6] += q

    @pl.when(p == 2)
    def _():
        hs = _h0_list(xc_ref, xk_ref, w0b, w0c, b0)
        hs = _layer(hs, acc[:, 0:1], acc[:, 1:2], g0, be0, w1, b1)
        hs = _layer(hs, acc[:, 4:5], acc[:, 5:6], g1, be1, w2, b2)
        s, q = _local_sums(hs)
        acc[:, 6:7] += s
        acc[:, 7:8] += q

    @pl.when(p == 3)
    def _():
        hs = _h0_list(xc_ref, xk_ref, w0b, w0c, b0)
        hs = _layer(hs, acc[:, 0:1], acc[:, 1:2], g0, be0, w1, b1)
        hs = _layer(hs, acc[:, 4:5], acc[:, 5:6], g1, be1, w2, b2)
        a2, c2 = _bn_const(acc[:, 6:7], acc[:, 7:8], g2, be2, NK_CNT)
        agg = jnp.zeros((C, RM), jnp.float32)
        for h in hs:
            agg += jnp.maximum(a2 * h + c2, 0.0)
        aggr = agg * jnp.float32(1.0 / K)
        scl = _dot(wsc[...], xc_ref[...]) + bsc[...]
        asc, csc = _bn_const(acc[:, 2:3], acc[:, 3:4], gsc, besc, NSC_CNT)
        out_ref[...] = jnp.maximum(aggr + asc * scl + csc, 0.0)


def _cspec(shape):
    return pl.BlockSpec(shape, lambda p, i: tuple(0 for _ in shape))


def kernel(x, W_sc, b_sc, g_sc, be_sc, W0, b0, g0, be0,
           W1, b1, g1, be1, W2, b2, g2, be2):
    xt = jnp.transpose(x.reshape(B * N, D))
    idx = _knn_call(x, xt)
    xk3 = _gather_call(x, idx)

    w0b = W0[:, D:]
    w0c = W0[:, :D] - W0[:, D:]
    col = lambda v: v.reshape(C, 1)
    b0c, bscc = col(b0), col(b_sc)
    b1c, b2c = col(b1), col(b2)
    g0c, be0c = col(g0), col(be0)
    g1c, be1c = col(g1), col(be1)
    g2c, be2c = col(g2), col(be2)
    gscc, bescc = col(g_sc), col(be_sc)

    row_specs = [
        pl.BlockSpec((D, RM), lambda p, i: (0, i)),
        pl.BlockSpec((K * D, RM), lambda p, i: (0, i)),
    ]
    wdc = _cspec((C, D))
    wcc = _cspec((C, C))
    vsp = _cspec((C, 1))

    out_t = pl.pallas_call(
        _mlp_body,
        grid=(4, B * N // RM),
        in_specs=row_specs + [wdc, wdc, vsp, vsp, vsp, wcc, vsp, vsp, vsp,
                              wcc, vsp, vsp, vsp, wdc, vsp, vsp, vsp],
        out_specs=pl.BlockSpec((C, RM), lambda p, i: (0, i)),
        out_shape=jax.ShapeDtypeStruct((C, B * N), jnp.float32),
        scratch_shapes=[pltpu.VMEM((C, 8), jnp.float32)],
    )(xt, xk3, w0b, w0c, b0c, g0c, be0c, W1, b1c, g1c, be1c,
      W2, b2c, g2c, be2c, W_sc, bscc, gscc, bescc)

    return jnp.transpose(out_t).reshape(B, N, C)


# speedup vs baseline: 1.4215x; 1.0549x over previous
import functools

import jax
import jax.numpy as jnp
from jax import lax
from jax.experimental import pallas as pl
from jax.experimental.pallas import tpu as pltpu
from jax.experimental.pallas import tpu_sc as plsc

B, N, D, K, C = 16, 1024, 16, 20, 64
EPS = 1e-5
NKTOT = B * N * K
NK_CNT = float(B * N * K)
NSC_CNT = float(B * N)
RN = 512
RM = 1024
LANES = 128
NROWS = NKTOT // LANES
NW = 32
CHUNKS_PER_W = NROWS // NW
PTS_PER_ROW = LANES // D


def _knn_body(xq_ref, xt_ref, idx_ref):
    b = pl.program_id(0)
    xi_p = xq_ref[0, :, 0:1]
    xi_e = xq_ref[0, :, 1:2]
    xj_p = xt_ref[0:1, :]
    xj_e = xt_ref[1:2, :]
    dx = xj_p - xi_p
    dy = xj_e - xi_e
    s = dx * dx + dy * dy
    jota = lax.broadcasted_iota(jnp.int32, (RN, N), 1)
    key = lax.bitcast_convert_type(
        ((lax.bitcast_convert_type(s, jnp.int32) & ~(N - 1)) | jota)
        + jnp.int32(0x00800000),
        jnp.float32)
    big = jnp.float32(3.0e38)
    cols = []
    for k in range(K):
        mk = jnp.min(key, axis=1, keepdims=True)
        cols.append(mk)
        if k + 1 < K:
            key = jnp.where(key == mk, big, key)
    mat = lax.bitcast_convert_type(
        jnp.concatenate(cols, axis=1), jnp.int32)
    idx_ref[...] = jnp.transpose((mat & (N - 1)) + b * N)


def _knn_call(x, xt):
    return pl.pallas_call(
        _knn_body,
        grid=(B, N // RN),
        in_specs=[
            pl.BlockSpec((1, RN, D), lambda b, r: (b, r, 0)),
            pl.BlockSpec((8, N), lambda b, r: (0, b)),
        ],
        out_specs=pl.BlockSpec((K, RN), lambda b, r: (0, b * (N // RN) + r)),
        out_shape=jax.ShapeDtypeStruct((K, B * N), jnp.int32),
    )(x, xt)


CH = 512


def _gather_body(xflat_hbm, idx_hbm, out_hbm,
                 x_tile, idx_v, p0, p1, sem0, sem1):
    wid = lax.axis_index("s") * 2 + lax.axis_index("c")
    b = wid // 2
    col0 = wid * CH
    pltpu.sync_copy(xflat_hbm.at[pl.ds(b * N * D, N * D)], x_tile)

    def gather_one(t, pk):
        pltpu.sync_copy(idx_hbm.at[t, pl.ds(col0, CH)], idx_v)
        for c in range(CH // 16):
            iv = idx_v[pl.ds(16 * c, 16)] & (N - 1)
            fbase = iv << 4
            vals = [plsc.load_gather(x_tile, [fbase + d]) for d in range(D)]
            for d in range(D):
                pk[d, pl.ds(16 * c, 16)] = vals[d]

    def pair(u, carry):
        for h, (pk, sem) in enumerate(((p0, sem0), (p1, sem1))):
            t = u * 2 + h

            @pl.when(u >= 1)
            def _():
                pltpu.make_async_copy(
                    pk, out_hbm.at[pl.ds(0, D), pl.ds(col0, CH)], sem).wait()

            gather_one(t, pk)
            pltpu.async_copy(
                pk, out_hbm.at[pl.ds(t * D, D), pl.ds(col0, CH)], sem)
        return carry

    lax.fori_loop(0, K // 2, pair, 0)
    pltpu.make_async_copy(
        p0, out_hbm.at[pl.ds(0, D), pl.ds(col0, CH)], sem0).wait()
    pltpu.make_async_copy(
        p1, out_hbm.at[pl.ds(0, D), pl.ds(col0, CH)], sem1).wait()


def _gather_call(x, idx):
    xflat = x.reshape(B * N * D)
    mesh = plsc.VectorSubcoreMesh(core_axis_name="c", subcore_axis_name="s")
    fn = functools.partial(
        pl.kernel,
        mesh=mesh,
        compiler_params=pltpu.CompilerParams(needs_layout_passes=False),
        out_type=jax.ShapeDtypeStruct((K * D, B * N), jnp.float32),
        scratch_types=[
            pltpu.VMEM((N * D,), jnp.float32),
            pltpu.VMEM((CH,), jnp.int32),
            pltpu.VMEM((D, CH), jnp.float32),
            pltpu.VMEM((D, CH), jnp.float32),
            pltpu.SemaphoreType.DMA,
            pltpu.SemaphoreType.DMA,
        ],
    )(_gather_body)
    return fn(xflat, idx)


def _bn_const(s, q, g_ref, be_ref, cnt):
    mean = s / cnt
    var = q / cnt - mean * mean
    a = g_ref[...] * lax.rsqrt(var + EPS)
    c = be_ref[...] - a * mean
    return a, c


def _dot(a, b):
    return jnp.dot(a.astype(jnp.bfloat16), b.astype(jnp.bfloat16),
                   preferred_element_type=jnp.float32)


def _h0_list(xc_ref, xk_ref, w0b_ref, w0c_ref, b0_ref):
    hc = _dot(w0c_ref[...], xc_ref[...]) + b0_ref[...]
    out = []
    for k in range(K):
        xkk = xk_ref[k * D:(k + 1) * D, :]
        out.append(_dot(w0b_ref[...], xkk) + hc)
    return out


def _local_sums(hs):
    s = jnp.zeros((C, 1), jnp.float32)
    q = jnp.zeros((C, 1), jnp.float32)
    for h in hs:
        s += jnp.sum(h, axis=1, keepdims=True)
        q += jnp.sum(h * h, axis=1, keepdims=True)
    return s, q


def _layer(hs, s, q, g, be, w, bias):
    a, c = _bn_const(s, q, g, be, NK_CNT)
    out = []
    for h in hs:
        r = jnp.maximum(a * h + c, 0.0)
        out.append(_dot(w[...], r) + bias[...])
    return out


def _mlp_body(xc_ref, xk_ref, w0b, w0c, b0, g0, be0, w1, b1, g1, be1,
              w2, b2, g2, be2, wsc, bsc, gsc, besc, out_ref, acc):
    p = pl.program_id(0)

    @pl.when((p == 0) & (pl.program_id(1) == 0))
    def _():
        acc[...] = jnp.zeros_like(acc)

    @pl.when(p == 0)
    def _():
        hs = _h0_list(xc_ref, xk_ref, w0b, w0c, b0)
        s, q = _local_sums(hs)
        scl = _dot(wsc[...], xc_ref[...]) + bsc[...]
        acc[:, 0:1] += s
        acc[:, 1:2] += q
        acc[:, 2:3] += jnp.sum(scl, axis=1, keepdims=True)
        acc[:, 3:4] += jnp.sum(scl * scl, axis=1, keepdims=True)

    @pl.when(p == 1)
    def _():
        hs = _h0_list(xc_ref, xk_ref, w0b, w0c, b0)
        hs = _layer(hs, acc[:, 0:1], acc[:, 1:2], g0, be0, w1, b1)
        s, q = _local_sums(hs)
        acc[:, 4:5] += s
        acc[:, 5:6] += q

    @pl.when(p == 2)
    def _():
        hs = _h0_list(xc_ref, xk_ref, w0b, w0c, b0)
        hs = _layer(hs, acc[:, 0:1], acc[:, 1:2], g0, be0, w1, b1)
        hs = _layer(hs, acc[:, 4:5], acc[:, 5:6], g1, be1, w2, b2)
        s, q = _local_sums(hs)
        acc[:, 6:7] += s
        acc[:, 7:8] += q

    @pl.when(p == 3)
    def _():
        hs = _h0_list(xc_ref, xk_ref, w0b, w0c, b0)
        hs = _layer(hs, acc[:, 0:1], acc[:, 1:2], g0, be0, w1, b1)
        hs = _layer(hs, acc[:, 4:5], acc[:, 5:6], g1, be1, w2, b2)
        a2, c2 = _bn_const(acc[:, 6:7], acc[:, 7:8], g2, be2, NK_CNT)
        agg = jnp.zeros((C, RM), jnp.float32)
        for h in hs:
            agg += jnp.maximum(a2 * h + c2, 0.0)
        aggr = agg * jnp.float32(1.0 / K)
        scl = _dot(wsc[...], xc_ref[...]) + bsc[...]
        asc, csc = _bn_const(acc[:, 2:3], acc[:, 3:4], gsc, besc, NSC_CNT)
        out_ref[...] = jnp.maximum(aggr + asc * scl + csc, 0.0)


def _cspec(shape):
    return pl.BlockSpec(shape, lambda p, i: tuple(0 for _ in shape))


def kernel(x, W_sc, b_sc, g_sc, be_sc, W0, b0, g0, be0,
           W1, b1, g1, be1, W2, b2, g2, be2):
    xt = jnp.transpose(x.reshape(B * N, D))
    idx = _knn_call(x, xt)
    xk3 = _gather_call(x, idx)

    w0b = W0[:, D:]
    w0c = W0[:, :D] - W0[:, D:]
    col = lambda v: v.reshape(C, 1)
    b0c, bscc = col(b0), col(b_sc)
    b1c, b2c = col(b1), col(b2)
    g0c, be0c = col(g0), col(be0)
    g1c, be1c = col(g1), col(be1)
    g2c, be2c = col(g2), col(be2)
    gscc, bescc = col(g_sc), col(be_sc)

    row_specs = [
        pl.BlockSpec((D, RM), lambda p, i: (0, i)),
        pl.BlockSpec((K * D, RM), lambda p, i: (0, i)),
    ]
    wdc = _cspec((C, D))
    wcc = _cspec((C, C))
    vsp = _cspec((C, 1))

    out_t = pl.pallas_call(
        _mlp_body,
        grid=(4, B * N // RM),
        in_specs=row_specs + [wdc, wdc, vsp, vsp, vsp, wcc, vsp, vsp, vsp,
                              wcc, vsp, vsp, vsp, wdc, vsp, vsp, vsp],
        out_specs=pl.BlockSpec((C, RM), lambda p, i: (0, i)),
        out_shape=jax.ShapeDtypeStruct((C, B * N), jnp.float32),
        scratch_shapes=[pltpu.VMEM((C, 8), jnp.float32)],
    )(xt, xk3, w0b, w0c, b0c, g0c, be0c, W1, b1c, g1c, be1c,
      W2, b2c, g2c, be2c, W_sc, bscc, gscc, bescc)

    return jnp.transpose(out_t).reshape(B, N, C)


# speedup vs baseline: 1.4269x; 1.0038x over previous
import functools

import jax
import jax.numpy as jnp
from jax import lax
from jax.experimental import pallas as pl
from jax.experimental.pallas import tpu as pltpu
from jax.experimental.pallas import tpu_sc as plsc

B, N, D, K, C = 16, 1024, 16, 20, 64
EPS = 1e-5
NKTOT = B * N * K
NK_CNT = float(B * N * K)
NSC_CNT = float(B * N)
RN = 512
RM = 1024
LANES = 128
NROWS = NKTOT // LANES
NW = 32
CHUNKS_PER_W = NROWS // NW
PTS_PER_ROW = LANES // D


def _knn_body(xq_ref, xt_ref, idx_ref):
    b = pl.program_id(0)
    xi_p = xq_ref[0, :, 0:1]
    xi_e = xq_ref[0, :, 1:2]
    xj_p = xt_ref[0:1, :]
    xj_e = xt_ref[1:2, :]
    dx = xj_p - xi_p
    dy = xj_e - xi_e
    s = dx * dx + dy * dy
    jota = lax.broadcasted_iota(jnp.int32, (RN, N), 1)
    key = lax.bitcast_convert_type(
        ((lax.bitcast_convert_type(s, jnp.int32) & ~(N - 1)) | jota)
        + jnp.int32(0x00800000),
        jnp.float32)
    big = jnp.float32(3.0e38)
    cols = []
    for k in range(K):
        mk = jnp.min(key, axis=1, keepdims=True)
        cols.append(mk)
        if k + 1 < K:
            key = jnp.where(key == mk, big, key)
    mat = lax.bitcast_convert_type(
        jnp.concatenate(cols, axis=1), jnp.int32)
    idx_ref[...] = jnp.transpose((mat & (N - 1)) + b * N)


def _knn_call(x, xt):
    return pl.pallas_call(
        _knn_body,
        grid=(B, N // RN),
        in_specs=[
            pl.BlockSpec((1, RN, D), lambda b, r: (b, r, 0)),
            pl.BlockSpec((8, N), lambda b, r: (0, b)),
        ],
        out_specs=pl.BlockSpec((K, RN), lambda b, r: (0, b * (N // RN) + r)),
        out_shape=jax.ShapeDtypeStruct((K, B * N), jnp.int32),
    )(x, xt)


CH = 512


def _gather_body(xflat_hbm, idx_hbm, out_hbm,
                 x_tile, idx_v, p0, p1, sem0, sem1):
    wid = lax.axis_index("s") * 2 + lax.axis_index("c")
    b = wid // 2
    col0 = wid * CH
    pltpu.sync_copy(xflat_hbm.at[pl.ds(b * N * D, N * D)], x_tile)

    def gather_one(t, pk):
        pltpu.sync_copy(idx_hbm.at[t, pl.ds(col0, CH)], idx_v)
        for c in range(CH // 16):
            iv = idx_v[pl.ds(16 * c, 16)] & (N - 1)
            fbase = iv << 4
            vals = [plsc.load_gather(x_tile, [fbase + d]) for d in range(D)]
            for d in range(D):
                pk[d, pl.ds(16 * c, 16)] = vals[d]

    def pair(u, carry):
        for h, (pk, sem) in enumerate(((p0, sem0), (p1, sem1))):
            t = u * 2 + h

            @pl.when(u >= 1)
            def _():
                pltpu.make_async_copy(
                    pk, out_hbm.at[pl.ds(0, D), pl.ds(col0, CH)], sem).wait()

            gather_one(t, pk)
            pltpu.async_copy(
                pk, out_hbm.at[pl.ds(t * D, D), pl.ds(col0, CH)], sem)
        return carry

    lax.fori_loop(0, K // 2, pair, 0)
    pltpu.make_async_copy(
        p0, out_hbm.at[pl.ds(0, D), pl.ds(col0, CH)], sem0).wait()
    pltpu.make_async_copy(
        p1, out_hbm.at[pl.ds(0, D), pl.ds(col0, CH)], sem1).wait()


def _gather_call(x, idx):
    xflat = x.reshape(B * N * D)
    mesh = plsc.VectorSubcoreMesh(core_axis_name="c", subcore_axis_name="s")
    fn = functools.partial(
        pl.kernel,
        mesh=mesh,
        compiler_params=pltpu.CompilerParams(needs_layout_passes=False),
        out_type=jax.ShapeDtypeStruct((K * D, B * N), jnp.float32),
        scratch_types=[
            pltpu.VMEM((N * D,), jnp.float32),
            pltpu.VMEM((CH,), jnp.int32),
            pltpu.VMEM((D, CH), jnp.float32),
            pltpu.VMEM((D, CH), jnp.float32),
            pltpu.SemaphoreType.DMA,
            pltpu.SemaphoreType.DMA,
        ],
    )(_gather_body)
    return fn(xflat, idx)


def _bn_const(s, q, g_ref, be_ref, cnt):
    mean = s / cnt
    var = q / cnt - mean * mean
    a = g_ref[...] * lax.rsqrt(var + EPS)
    c = be_ref[...] - a * mean
    return a, c


def _dot(a, b):
    return jnp.dot(a.astype(jnp.bfloat16), b.astype(jnp.bfloat16),
                   preferred_element_type=jnp.float32)


def _h0_list(xc_ref, xk_ref, w0b_ref, w0c_ref, b0_ref):
    hc = _dot(w0c_ref[...], xc_ref[...]) + b0_ref[...]
    out = []
    for k in range(K):
        xkk = xk_ref[k * D:(k + 1) * D, :]
        out.append(_dot(w0b_ref[...], xkk) + hc)
    return out


def _local_sums(hs):
    s = jnp.zeros((C, 1), jnp.float32)
    q = jnp.zeros((C, 1), jnp.float32)
    for h in hs:
        s += jnp.sum(h, axis=1, keepdims=True)
        q += jnp.sum(h * h, axis=1, keepdims=True)
    return s, q


def _layer(hs, s, q, g, be, w, bias):
    a, c = _bn_const(s, q, g, be, NK_CNT)
    out = []
    for h in hs:
        r = jnp.maximum(a * h + c, 0.0)
        out.append(_dot(w[...], r) + bias[...])
    return out


def _relu_list(hs, s, q, g, be):
    a, c = _bn_const(s, q, g, be, NK_CNT)
    return [jnp.maximum(a * h + c, 0.0) for h in hs]


def _lin_stats(r_list, w_ref, b_ref):
    rc = jnp.concatenate(r_list, axis=1).astype(jnp.bfloat16)
    g_mat = lax.dot_general(rc, rc, (((1,), (1,)), ((), ())),
                            preferred_element_type=jnp.float32)
    rs = jnp.sum(jnp.concatenate(
        [jnp.sum(r, axis=1, keepdims=True) for r in r_list], axis=1),
        axis=1, keepdims=True)
    w = w_ref[...]
    b = b_ref[...]
    cnt = float(len(r_list) * RM)
    wr = jnp.dot(w, rs, preferred_element_type=jnp.float32)
    wg = jnp.dot(w, g_mat, preferred_element_type=jnp.float32)
    s = wr + cnt * b
    q = (jnp.sum(wg * w, axis=1, keepdims=True) + 2.0 * b * wr
         + cnt * b * b)
    return s, q


def _mlp_body(xc_ref, xk_ref, w0b, w0c, b0, g0, be0, w1, b1, g1, be1,
              w2, b2, g2, be2, wsc, bsc, gsc, besc, out_ref, acc):
    p = pl.program_id(0)

    @pl.when((p == 0) & (pl.program_id(1) == 0))
    def _():
        acc[...] = jnp.zeros_like(acc)

    @pl.when(p == 0)
    def _():
        hs = _h0_list(xc_ref, xk_ref, w0b, w0c, b0)
        s, q = _local_sums(hs)
        scl = _dot(wsc[...], xc_ref[...]) + bsc[...]
        acc[:, 0:1] += s
        acc[:, 1:2] += q
        acc[:, 2:3] += jnp.sum(scl, axis=1, keepdims=True)
        acc[:, 3:4] += jnp.sum(scl * scl, axis=1, keepdims=True)

    @pl.when(p == 1)
    def _():
        hs = _h0_list(xc_ref, xk_ref, w0b, w0c, b0)
        rs = _relu_list(hs, acc[:, 0:1], acc[:, 1:2], g0, be0)
        s, q = _lin_stats(rs, w1, b1)
        acc[:, 4:5] += s
        acc[:, 5:6] += q

    @pl.when(p == 2)
    def _():
        hs = _h0_list(xc_ref, xk_ref, w0b, w0c, b0)
        hs = _layer(hs, acc[:, 0:1], acc[:, 1:2], g0, be0, w1, b1)
        rs = _relu_list(hs, acc[:, 4:5], acc[:, 5:6], g1, be1)
        s, q = _lin_stats(rs, w2, b2)
        acc[:, 6:7] += s
        acc[:, 7:8] += q

    @pl.when(p == 3)
    def _():
        hs = _h0_list(xc_ref, xk_ref, w0b, w0c, b0)
        hs = _layer(hs, acc[:, 0:1], acc[:, 1:2], g0, be0, w1, b1)
        hs = _layer(hs, acc[:, 4:5], acc[:, 5:6], g1, be1, w2, b2)
        a2, c2 = _bn_const(acc[:, 6:7], acc[:, 7:8], g2, be2, NK_CNT)
        agg = jnp.zeros((C, RM), jnp.float32)
        for h in hs:
            agg += jnp.maximum(a2 * h + c2, 0.0)
        aggr = agg * jnp.float32(1.0 / K)
        scl = _dot(wsc[...], xc_ref[...]) + bsc[...]
        asc, csc = _bn_const(acc[:, 2:3], acc[:, 3:4], gsc, besc, NSC_CNT)
        out_ref[...] = jnp.maximum(aggr + asc * scl + csc, 0.0)


def _cspec(shape):
    return pl.BlockSpec(shape, lambda p, i: tuple(0 for _ in shape))


def kernel(x, W_sc, b_sc, g_sc, be_sc, W0, b0, g0, be0,
           W1, b1, g1, be1, W2, b2, g2, be2):
    xt = jnp.transpose(x.reshape(B * N, D))
    idx = _knn_call(x, xt)
    xk3 = _gather_call(x, idx)

    w0b = W0[:, D:]
    w0c = W0[:, :D] - W0[:, D:]
    col = lambda v: v.reshape(C, 1)
    b0c, bscc = col(b0), col(b_sc)
    b1c, b2c = col(b1), col(b2)
    g0c, be0c = col(g0), col(be0)
    g1c, be1c = col(g1), col(be1)
    g2c, be2c = col(g2), col(be2)
    gscc, bescc = col(g_sc), col(be_sc)

    row_specs = [
        pl.BlockSpec((D, RM), lambda p, i: (0, i)),
        pl.BlockSpec((K * D, RM), lambda p, i: (0, i)),
    ]
    wdc = _cspec((C, D))
    wcc = _cspec((C, C))
    vsp = _cspec((C, 1))

    out_t = pl.pallas_call(
        _mlp_body,
        grid=(4, B * N // RM),
        in_specs=row_specs + [wdc, wdc, vsp, vsp, vsp, wcc, vsp, vsp, vsp,
                              wcc, vsp, vsp, vsp, wdc, vsp, vsp, vsp],
        out_specs=pl.BlockSpec((C, RM), lambda p, i: (0, i)),
        out_shape=jax.ShapeDtypeStruct((C, B * N), jnp.float32),
        scratch_shapes=[pltpu.VMEM((C, 8), jnp.float32)],
    )(xt, xk3, w0b, w0c, b0c, g0c, be0c, W1, b1c, g1c, be1c,
      W2, b2c, g2c, be2c, W_sc, bscc, gscc, bescc)

    return jnp.transpose(out_t).reshape(B, N, C)
